# Initial kernel scaffold; baseline (speedup 1.0000x reference)
import jax
import jax.numpy as jnp
from jax.experimental import pallas as pl


def kernel(x, hms):
    raise NotImplementedError("write your pallas kernel here")


# speedup vs baseline: 1648.9316x; 1648.9316x over previous
import functools

import jax
import jax.numpy as jnp
from jax import lax
from jax.experimental import pallas as pl
from jax.experimental.pallas import tpu as pltpu
from jax.experimental.pallas import tpu_sc as plsc

L = 16
NC = 2
NS = 16
NW = NC * NS

N_ELEMS = 2 * 4096 * 4096
PER_W = N_ELEMS // NW
CHUNK = 32768
NCHUNK = PER_W // CHUNK


def _sc_partial_sums(xf, hms):
    mesh = plsc.VectorSubcoreMesh(core_axis_name="c", subcore_axis_name="s")

    @functools.partial(
        pl.kernel,
        mesh=mesh,
        compiler_params=pltpu.CompilerParams(needs_layout_passes=False),
        out_type=jax.ShapeDtypeStruct((NW, L), jnp.float32),
        scratch_types=[
            pltpu.VMEM((2, CHUNK), jnp.float32),
            pltpu.VMEM((256,), jnp.float32),
            pltpu.VMEM((L,), jnp.float32),
            pltpu.SemaphoreType.DMA,
            pltpu.SemaphoreType.DMA,
        ],
    )
    def k(x_hbm, hms_hbm, out_hbm, buf, hms_v, res_v, sem0, sem1):
        wid = lax.axis_index("s") * NC + lax.axis_index("c")
        base = wid * PER_W
        pltpu.sync_copy(hms_hbm, hms_v)

        sems = (sem0, sem1)
        copies = [None, None]
        copies[0] = pltpu.async_copy(
            x_hbm.at[pl.ds(base, CHUNK)], buf.at[0], sems[0])

        def body(i, acc, slot):
            v = buf[slot, pl.ds(i * L, L)]
            y = jnp.clip(v + 128.0, 0.0, 255.0)
            low = y.astype(jnp.int32)
            frac = y - low.astype(jnp.float32)
            high = jnp.minimum(low + 1, 255)
            lv = plsc.load_gather(hms_v, [low])
            hv = plsc.load_gather(hms_v, [high])
            return acc + (lv + frac * (hv - lv))

        acc = jnp.zeros((L,), jnp.float32)
        for c in range(NCHUNK):
            slot = c % 2
            copies[slot].wait()
            if c + 1 < NCHUNK:
                nslot = (c + 1) % 2
                copies[nslot] = pltpu.async_copy(
                    x_hbm.at[pl.ds(base + (c + 1) * CHUNK, CHUNK)],
                    buf.at[nslot], sems[nslot])
            acc = lax.fori_loop(
                0, CHUNK // L, functools.partial(body, slot=slot), acc)

        res_v[...] = acc
        pltpu.sync_copy(res_v, out_hbm.at[wid])

    return k(xf, hms)


def _tc_reduce(partials):
    def rk(p_ref, o_ref):
        o_ref[0, 0] = jnp.sum(p_ref[...])

    return pl.pallas_call(
        rk,
        out_shape=jax.ShapeDtypeStruct((1, 1), jnp.float32),
        out_specs=pl.BlockSpec(memory_space=pltpu.SMEM),
    )(partials)


def kernel(x, hms):
    xf = x.reshape(-1)
    partials = _sc_partial_sums(xf, hms)
    total = _tc_reduce(partials)
    return total[0, 0]


# speedup vs baseline: 2243.8940x; 1.3608x over previous
import functools

import jax
import jax.numpy as jnp
from jax import lax
from jax.experimental import pallas as pl
from jax.experimental.pallas import tpu as pltpu
from jax.experimental.pallas import tpu_sc as plsc

L = 16
NC = 2
NS = 16
NW = NC * NS

N_ELEMS = 2 * 4096 * 4096
PER_W = N_ELEMS // NW
CHUNK = 32768
NCHUNK = PER_W // CHUNK
U = 4


def _sc_partial_sums(xf, hms):
    mesh = plsc.VectorSubcoreMesh(core_axis_name="c", subcore_axis_name="s")

    @functools.partial(
        pl.kernel,
        mesh=mesh,
        compiler_params=pltpu.CompilerParams(needs_layout_passes=False),
        out_type=jax.ShapeDtypeStruct((NW, L), jnp.float32),
        scratch_types=[
            pltpu.VMEM((2, CHUNK), jnp.float32),
            pltpu.VMEM((256,), jnp.float32),
            pltpu.VMEM((256,), jnp.float32),
            pltpu.VMEM((L,), jnp.float32),
            pltpu.SemaphoreType.DMA,
            pltpu.SemaphoreType.DMA,
        ],
    )
    def k(x_hbm, hms_hbm, out_hbm, buf, lut_lo, lut_hi, res_v, sem0, sem1):
        wid = lax.axis_index("s") * NC + lax.axis_index("c")
        base = wid * PER_W
        pltpu.sync_copy(hms_hbm, lut_lo)
        lane = lax.iota(jnp.int32, L)
        for j in range(256 // L):
            idx = jnp.minimum(lane + (j * L + 1), 255)
            lut_hi[pl.ds(j * L, L)] = plsc.load_gather(lut_lo, [idx])

        sems = (sem0, sem1)
        copies = [None, None]
        copies[0] = pltpu.async_copy(
            x_hbm.at[pl.ds(base, CHUNK)], buf.at[0], sems[0])

        def body(i, accs, slot):
            new = []
            for u in range(U):
                v = buf[slot, pl.ds((i * U + u) * L, L)]
                y = jnp.minimum(jnp.maximum(v + 128.0, 0.0), 255.0)
                low = y.astype(jnp.int32)
                frac = y - low.astype(jnp.float32)
                lv = plsc.load_gather(lut_lo, [low])
                hv = plsc.load_gather(lut_hi, [low])
                new.append(accs[u] + (lv + frac * (hv - lv)))
            return tuple(new)

        accs = (jnp.zeros((L,), jnp.float32),) * U
        for c in range(NCHUNK):
            slot = c % 2
            copies[slot].wait()
            if c + 1 < NCHUNK:
                nslot = (c + 1) % 2
                copies[nslot] = pltpu.async_copy(
                    x_hbm.at[pl.ds(base + (c + 1) * CHUNK, CHUNK)],
                    buf.at[nslot], sems[nslot])
            accs = plsc.parallel_loop(
                0, CHUNK // (L * U), 1, unroll=2, carry=accs)(
                    functools.partial(body, slot=slot))

        total = accs[0] + accs[1]
        if U > 2:
            for u in range(2, U):
                total = total + accs[u]
        res_v[...] = total
        pltpu.sync_copy(res_v, out_hbm.at[wid])

    return k(xf, hms)


def _tc_reduce(partials):
    def rk(p_ref, o_ref):
        o_ref[0, 0] = jnp.sum(p_ref[...])

    return pl.pallas_call(
        rk,
        out_shape=jax.ShapeDtypeStruct((1, 1), jnp.float32),
        out_specs=pl.BlockSpec(memory_space=pltpu.SMEM),
    )(partials)


def kernel(x, hms):
    xf = x.reshape(-1)
    partials = _sc_partial_sums(xf, hms)
    total = _tc_reduce(partials)
    return total[0, 0]
